# baseline (speedup 1.0000x reference)
import functools

import jax
import jax.numpy as jnp
from jax import lax
from jax.experimental import pallas as pl
from jax.experimental.pallas import tpu as pltpu
from jax.experimental.pallas import tpu_sc as plsc

_N = 10000
_E = 320000
_F = 128
_FH = _F // 2
_NC = 2
_NS = 16
_NW = _NC * _NS
_EPW = _E // _NW
_EPT = _E // _NS
_CH = 80
_NCHUNK = _EPT // _CH
_RPT = _N // _NS
_NR = 640

_mesh = plsc.VectorSubcoreMesh(core_axis_name="c", subcore_axis_name="s")
_sc_params = pltpu.CompilerParams(needs_layout_passes=False,
                                  use_tc_tiling_on_sc=False)


@functools.partial(
    pl.kernel,
    out_type=jax.ShapeDtypeStruct((2, _E), jnp.float32),
    mesh=_mesh,
    compiler_params=_sc_params,
    scratch_types=[
        pltpu.VMEM((_EPT,), jnp.int32),
        pltpu.VMEM((_EPT,), jnp.float32),
        pltpu.VMEM((_NR, 16), jnp.float32),
        pltpu.VMEM((_EPT,), jnp.float32),
        pltpu.VMEM((5, 128), jnp.int32),
        pltpu.VMEM_SHARED((_NR, 16), jnp.float32),
    ],
)
def _degscale_kernel(row_hbm, col_hbm, wgt_hbm, out_hbm,
                     nod_v, wgt_v, deg_v, scl_v, idx_v, sdeg):
    cid = lax.axis_index("c")
    sid = lax.axis_index("s")

    @pl.when(cid == 0)
    def _load_row():
        pltpu.sync_copy(row_hbm.at[pl.ds(sid * _EPT, _EPT)], nod_v)

    @pl.when(cid == 1)
    def _load_col():
        pltpu.sync_copy(col_hbm.at[pl.ds(sid * _EPT, _EPT)], nod_v)

    pltpu.sync_copy(wgt_hbm.at[pl.ds(sid * _EPT, _EPT)], wgt_v)

    zeros16 = jnp.zeros((16,), jnp.float32)
    one16 = jnp.full((16,), 1.0, jnp.float32)
    iota16 = lax.iota(jnp.int32, 16)

    def _zero(i, carry):
        deg_v[i, pl.ds(0, 16)] = zeros16
        return carry

    lax.fori_loop(0, _NR, _zero, 0)

    @pl.when(sid == 0)
    def _zero_shared():
        pltpu.sync_copy(deg_v, sdeg)

    for k in range(5):
        for u in range(8):
            idx_v[k, pl.ds(16 * u, 16)] = iota16 + (128 * k + 16 * u)

    def _acc(i, carry):
        n16 = nod_v[pl.ds(16 * i, 16)]
        w16 = wgt_v[pl.ds(16 * i, 16)]
        plsc.addupdate_scatter(
            deg_v,
            [lax.shift_right_logical(n16, 4), lax.bitwise_and(n16, 15)], w16)
        return carry

    lax.fori_loop(0, _EPT // 16, _acc, 0)

    plsc.subcore_barrier()

    for k in range(5):
        pltpu.sync_copy(deg_v.at[pl.ds(128 * k, 128)],
                        sdeg.at[idx_v.at[k]], add=True)

    plsc.subcore_barrier()

    pltpu.sync_copy(sdeg, deg_v)

    def _recip(i, carry):
        d = deg_v[i, pl.ds(0, 16)]
        deg_v[i, pl.ds(0, 16)] = one16 / jnp.where(d == 0.0, one16, d)
        return carry

    lax.fori_loop(0, _NR, _recip, 0)

    def _scale(i, carry):
        n16 = nod_v[pl.ds(16 * i, 16)]
        rd = plsc.load_gather(
            deg_v,
            [lax.shift_right_logical(n16, 4), lax.bitwise_and(n16, 15)])
        scl_v[pl.ds(16 * i, 16)] = wgt_v[pl.ds(16 * i, 16)] * rd
        return carry

    lax.fori_loop(0, _EPT // 16, _scale, 0)

    pltpu.sync_copy(scl_v, out_hbm.at[cid, pl.ds(sid * _EPT, _EPT)])


@functools.partial(
    pl.kernel,
    out_type=jax.ShapeDtypeStruct((4, _NC, _N, _FH), jnp.float32),
    mesh=_mesh,
    compiler_params=_sc_params,
    scratch_types=[
        pltpu.VMEM((_EPT,), jnp.int32),
        pltpu.VMEM((_NCHUNK, _CH), jnp.int32),
        pltpu.VMEM((_EPT,), jnp.float32),
        pltpu.VMEM((_CH, _FH), jnp.float32),
        pltpu.VMEM((_CH, _FH), jnp.float32),
        pltpu.VMEM((_CH, _FH), jnp.float32),
        pltpu.VMEM((_CH, _FH), jnp.float32),
        pltpu.VMEM_SHARED((_N, _FH), jnp.float32),
        pltpu.SemaphoreType.DMA,
        pltpu.SemaphoreType.DMA,
        pltpu.SemaphoreType.DMA,
        pltpu.SemaphoreType.DMA,
    ],
)
def _prop_kernel(xc, gsrc_hbm, sdst_hbm, scale_hbm,
                 out_hbm,
                 idx_v, dst_v, scale_v, buf0, buf1, buf2, buf3,
                 accum, sem0, sem1, sem2, sem3):
    cid = lax.axis_index("c")
    sid = lax.axis_index("s")
    bufs = [buf0, buf1, buf2, buf3]
    sems = [sem0, sem1, sem2, sem3]

    pltpu.sync_copy(gsrc_hbm.at[pl.ds(sid * _EPT, _EPT)], idx_v)
    pltpu.sync_copy(sdst_hbm.at[pl.ds(sid * _NCHUNK, _NCHUNK)], dst_v)
    pltpu.sync_copy(scale_hbm.at[pl.ds(sid * _EPT, _EPT)], scale_v)

    zeros16 = jnp.zeros((16,), jnp.float32)

    coff = lax.broadcast(cid * _N, (16,))

    def _coff(i, carry):
        idx_v[pl.ds(16 * i, 16)] = idx_v[pl.ds(16 * i, 16)] + coff
        return carry

    lax.fori_loop(0, _EPT // 16, _coff, 0)

    def _zb(r, carry):
        for f in range(_FH // 16):
            buf0[r, pl.ds(16 * f, 16)] = zeros16
        return carry

    lax.fori_loop(0, _CH, _zb, 0)

    row0 = sid * _RPT
    sizes = []
    off = 0
    while off < _RPT:
        sizes.append(min(_CH, _RPT - off))
        off += _CH

    def _start(j, buf, sem):
        pltpu.async_copy(
            xc.at[idx_v.at[pl.ds(j * _CH, _CH)]], buf, sem)

    def _wait(j, buf, sem):
        pltpu.make_async_copy(
            xc.at[idx_v.at[pl.ds(j * _CH, _CH)]], buf, sem).wait()

    def _proc(j, buf):
        def _row(r, carry):
            iv = lax.broadcast(j * _CH + r, (16,))
            sv = plsc.load_gather(scale_v, [iv])
            for f in range(_FH // 16):
                buf[r, pl.ds(16 * f, 16)] = buf[r, pl.ds(16 * f, 16)] * sv
            return carry

        lax.fori_loop(0, _CH, _row, 0)
        pltpu.sync_copy(buf, accum.at[dst_v.at[j]], add=True)

    def _batch(b, carry):
        off = 0
        for sz in sizes:
            pltpu.sync_copy(buf0.at[pl.ds(0, sz)],
                            accum.at[pl.ds(row0 + off, sz)])
            off += sz
        plsc.subcore_barrier()

        _start(0, bufs[0], sems[0])
        _start(1, bufs[1], sems[1])
        _start(2, bufs[2], sems[2])

        def _grp(g, carry2):
            j0 = 4 * g
            for s in range(4):
                j = j0 + s
                _wait(j, bufs[s], sems[s])
                _proc(j, bufs[s])
                _start(j + 3, bufs[(s + 3) % 4], sems[(s + 3) % 4])
            return carry2

        lax.fori_loop(0, _NCHUNK // 4 - 1, _grp, 0)

        jt = _NCHUNK - 6
        for (j, s, nxt) in ((jt, 0, jt + 3), (jt + 1, 1, jt + 4),
                            (jt + 2, 2, jt + 5), (jt + 3, 3, None),
                            (jt + 4, 0, None), (jt + 5, 1, None)):
            _wait(j, bufs[s], sems[s])
            _proc(j, bufs[s])
            if nxt is not None:
                _start(nxt, bufs[(s + 3) % 4], sems[(s + 3) % 4])

        boff = lax.broadcast(2 * _N, (16,))

        def _bump(i, c2):
            idx_v[pl.ds(16 * i, 16)] = idx_v[pl.ds(16 * i, 16)] + boff
            return c2

        lax.fori_loop(0, _EPT // 16, _bump, 0)

        plsc.subcore_barrier()

        off = 0
        for sz in sizes:
            pltpu.sync_copy(accum.at[pl.ds(row0 + off, sz)],
                            out_hbm.at[b, cid, pl.ds(row0 + off, sz)])
            off += sz
        lax.fori_loop(0, _CH, _zb, 0)
        return carry

    lax.fori_loop(0, 4, _batch, 0)


_BLK = 400


def _gate_body(x_r, po_r, pi_r, wc_r, bc_r, y_r):
    X = x_r[0]
    acc = jnp.dot(X, wc_r[0:_F], preferred_element_type=jnp.float32)
    acc += jnp.dot(po_r[0, 0], wc_r[_F:_F + _FH],
                   preferred_element_type=jnp.float32)
    acc += jnp.dot(po_r[0, 1], wc_r[_F + _FH:2 * _F],
                   preferred_element_type=jnp.float32)
    acc += jnp.dot(pi_r[0, 0], wc_r[2 * _F:2 * _F + _FH],
                   preferred_element_type=jnp.float32)
    acc += jnp.dot(pi_r[0, 1], wc_r[2 * _F + _FH:3 * _F],
                   preferred_element_type=jnp.float32)
    acc += bc_r[...]
    z = jax.nn.sigmoid(acc[:, :_F])
    h = jnp.tanh(acc[:, _F:])
    y_r[0] = (1.0 - z) * h


def _gate(x, po, pi, wc, bc):
    grid = (4, _N // _BLK)
    return pl.pallas_call(
        _gate_body,
        grid=grid,
        in_specs=[
            pl.BlockSpec((1, _BLK, _F), lambda b, n: (b, n, 0)),
            pl.BlockSpec((1, _NC, _BLK, _FH), lambda b, n: (b, 0, n, 0)),
            pl.BlockSpec((1, _NC, _BLK, _FH), lambda b, n: (b, 0, n, 0)),
            pl.BlockSpec((3 * _F, 2 * _F), lambda b, n: (0, 0)),
            pl.BlockSpec((1, 2 * _F), lambda b, n: (0, 0)),
        ],
        out_specs=pl.BlockSpec((1, _BLK, _F), lambda b, n: (b, n, 0)),
        out_shape=jax.ShapeDtypeStruct((4, _N, _F), jnp.float32),
    )(x, po, pi, wc, bc)


def kernel(x, edge_index, edge_weight, W_z, b_z, W_r, b_r, W_h, b_h):
    row = edge_index[0]
    col = edge_index[1]

    scales = _degscale_kernel(row, col, edge_weight)

    xc = jnp.concatenate(
        [jnp.concatenate([x[b, :, :_FH], x[b, :, _FH:]], axis=0)
         for b in range(4)], axis=0)
    col2 = col.reshape(_E // _CH, _CH)
    row2 = row.reshape(_E // _CH, _CH)

    po = _prop_kernel(xc, row, col2, scales[0])
    pi = _prop_kernel(xc, col, row2, scales[1])

    def _fold(W):
        return jnp.concatenate(
            [W[0, 0, :_F] + W[1, 0, :_F], W[0, 1, :_F], W[1, 1, :_F]], axis=0)

    wc = jnp.concatenate([_fold(W_z), _fold(W_h)], axis=1)
    bc = jnp.concatenate([b_z, b_h]).reshape(1, 2 * _F)

    return _gate(x, po, pi, wc, bc)


# speedup vs baseline: 1.0665x; 1.0665x over previous
import functools

import jax
import jax.numpy as jnp
from jax import lax
from jax.experimental import pallas as pl
from jax.experimental.pallas import tpu as pltpu
from jax.experimental.pallas import tpu_sc as plsc

_N = 10000
_E = 320000
_F = 128
_FH = _F // 2
_NC = 2
_NS = 16
_EPT = _E // _NS
_CH = 80
_NCHUNK = _EPT // _CH
_RPT = _N // _NS
_NR = 640

_mesh = plsc.VectorSubcoreMesh(core_axis_name="c", subcore_axis_name="s")
_sc_params = pltpu.CompilerParams(needs_layout_passes=False,
                                  use_tc_tiling_on_sc=False)


@functools.partial(
    pl.kernel,
    out_type=jax.ShapeDtypeStruct((2, _E), jnp.float32),
    mesh=_mesh,
    compiler_params=_sc_params,
    scratch_types=[
        pltpu.VMEM((_EPT,), jnp.int32),
        pltpu.VMEM((_EPT,), jnp.float32),
        pltpu.VMEM((_NR, 16), jnp.float32),
        pltpu.VMEM((_EPT,), jnp.float32),
        pltpu.VMEM((5, 128), jnp.int32),
        pltpu.VMEM_SHARED((_NR, 16), jnp.float32),
    ],
)
def _degscale_kernel(row_hbm, col_hbm, wgt_hbm, out_hbm,
                     nod_v, wgt_v, deg_v, scl_v, idx_v, sdeg):
    cid = lax.axis_index("c")
    sid = lax.axis_index("s")

    @pl.when(cid == 0)
    def _load_row():
        pltpu.sync_copy(row_hbm.at[pl.ds(sid * _EPT, _EPT)], nod_v)

    @pl.when(cid == 1)
    def _load_col():
        pltpu.sync_copy(col_hbm.at[pl.ds(sid * _EPT, _EPT)], nod_v)

    pltpu.sync_copy(wgt_hbm.at[pl.ds(sid * _EPT, _EPT)], wgt_v)

    zeros16 = jnp.zeros((16,), jnp.float32)
    one16 = jnp.full((16,), 1.0, jnp.float32)
    iota16 = lax.iota(jnp.int32, 16)

    def _zero(i, carry):
        deg_v[i, pl.ds(0, 16)] = zeros16
        return carry

    lax.fori_loop(0, _NR, _zero, 0)

    @pl.when(sid == 0)
    def _zero_shared():
        pltpu.sync_copy(deg_v, sdeg)

    for k in range(5):
        for u in range(8):
            idx_v[k, pl.ds(16 * u, 16)] = iota16 + (128 * k + 16 * u)

    def _acc(i, carry):
        n16 = nod_v[pl.ds(16 * i, 16)]
        w16 = wgt_v[pl.ds(16 * i, 16)]
        plsc.addupdate_scatter(
            deg_v,
            [lax.shift_right_logical(n16, 4), lax.bitwise_and(n16, 15)], w16)
        return carry

    lax.fori_loop(0, _EPT // 16, _acc, 0)

    plsc.subcore_barrier()

    for k in range(5):
        pltpu.sync_copy(deg_v.at[pl.ds(128 * k, 128)],
                        sdeg.at[idx_v.at[k]], add=True)

    plsc.subcore_barrier()

    pltpu.sync_copy(sdeg, deg_v)

    def _recip(i, carry):
        d = deg_v[i, pl.ds(0, 16)]
        deg_v[i, pl.ds(0, 16)] = one16 / jnp.where(d == 0.0, one16, d)
        return carry

    lax.fori_loop(0, _NR, _recip, 0)

    def _scale(i, carry):
        n16 = nod_v[pl.ds(16 * i, 16)]
        rd = plsc.load_gather(
            deg_v,
            [lax.shift_right_logical(n16, 4), lax.bitwise_and(n16, 15)])
        scl_v[pl.ds(16 * i, 16)] = wgt_v[pl.ds(16 * i, 16)] * rd
        return carry

    lax.fori_loop(0, _EPT // 16, _scale, 0)

    pltpu.sync_copy(scl_v, out_hbm.at[cid, pl.ds(sid * _EPT, _EPT)])


@functools.partial(
    pl.kernel,
    out_type=jax.ShapeDtypeStruct((4, _NC, _N, _FH), jnp.float32),
    mesh=_mesh,
    compiler_params=_sc_params,
    scratch_types=[
        pltpu.VMEM((_EPT,), jnp.int32),
        pltpu.VMEM((_NCHUNK, _CH), jnp.int32),
        pltpu.VMEM((_EPT,), jnp.float32),
        pltpu.VMEM((_CH, _FH), jnp.float32),
        pltpu.VMEM((_CH, _FH), jnp.float32),
        pltpu.VMEM_SHARED((_N, _FH), jnp.float32),
        pltpu.SemaphoreType.DMA,
        pltpu.SemaphoreType.DMA,
    ],
)
def _prop_kernel(xc, gsrc_hbm, sdst_hbm, scale_hbm,
                 out_hbm,
                 idx_v, dst_v, scale_v, buf0, buf1,
                 accum, sem0, sem1):
    cid = lax.axis_index("c")
    sid = lax.axis_index("s")

    pltpu.sync_copy(gsrc_hbm.at[pl.ds(sid * _EPT, _EPT)], idx_v)
    pltpu.sync_copy(sdst_hbm.at[pl.ds(sid * _NCHUNK, _NCHUNK)], dst_v)
    pltpu.sync_copy(scale_hbm.at[pl.ds(sid * _EPT, _EPT)], scale_v)

    zeros16 = jnp.zeros((16,), jnp.float32)

    coff = lax.broadcast(cid * _N, (16,))

    def _coff(i, carry):
        idx_v[pl.ds(16 * i, 16)] = idx_v[pl.ds(16 * i, 16)] + coff
        return carry

    lax.fori_loop(0, _EPT // 16, _coff, 0)

    def _zb(r, carry):
        for f in range(_FH // 16):
            buf0[r, pl.ds(16 * f, 16)] = zeros16
        return carry

    lax.fori_loop(0, _CH, _zb, 0)

    row0 = sid * _RPT
    sizes = []
    off = 0
    while off < _RPT:
        sizes.append(min(_CH, _RPT - off))
        off += _CH

    def _start(j, buf, sem):
        pltpu.async_copy(
            xc.at[idx_v.at[pl.ds(j * _CH, _CH)]], buf, sem)

    def _wait(j, buf, sem):
        pltpu.make_async_copy(
            xc.at[idx_v.at[pl.ds(j * _CH, _CH)]], buf, sem).wait()

    def _proc(j, buf):
        for g in range(_CH // 16):
            sc16 = scale_v[pl.ds(j * _CH + 16 * g, 16)]
            for rr in range(16):
                r = 16 * g + rr
                sv = lax.broadcast(sc16[rr], (16,))
                for f in range(_FH // 16):
                    buf[r, pl.ds(16 * f, 16)] = \
                        buf[r, pl.ds(16 * f, 16)] * sv
        pltpu.sync_copy(buf, accum.at[dst_v.at[j]], add=True)

    def _batch(b, carry):
        off = 0
        for sz in sizes:
            pltpu.sync_copy(buf0.at[pl.ds(0, sz)],
                            accum.at[pl.ds(row0 + off, sz)])
            off += sz
        plsc.subcore_barrier()

        _start(0, buf0, sem0)

        def _pair(j2, carry2):
            j = 2 * j2
            _wait(j, buf0, sem0)
            _start(j + 1, buf1, sem1)
            _proc(j, buf0)
            _wait(j + 1, buf1, sem1)
            _start(j + 2, buf0, sem0)
            _proc(j + 1, buf1)
            return carry2

        lax.fori_loop(0, _NCHUNK // 2 - 1, _pair, 0)

        j = _NCHUNK - 2
        _wait(j, buf0, sem0)
        _start(j + 1, buf1, sem1)
        _proc(j, buf0)
        _wait(j + 1, buf1, sem1)
        _proc(j + 1, buf1)

        boff = lax.broadcast(2 * _N, (16,))

        def _bump(i, c2):
            idx_v[pl.ds(16 * i, 16)] = idx_v[pl.ds(16 * i, 16)] + boff
            return c2

        lax.fori_loop(0, _EPT // 16, _bump, 0)

        plsc.subcore_barrier()

        off = 0
        for sz in sizes:
            pltpu.sync_copy(accum.at[pl.ds(row0 + off, sz)],
                            out_hbm.at[b, cid, pl.ds(row0 + off, sz)])
            off += sz
        lax.fori_loop(0, _CH, _zb, 0)
        return carry

    lax.fori_loop(0, 4, _batch, 0)


_BLK = 400


def _gate_body(x_r, po_r, pi_r, wc_r, bc_r, y_r):
    X = x_r[0]
    acc = jnp.dot(X, wc_r[0:_F], preferred_element_type=jnp.float32)
    acc += jnp.dot(po_r[0, 0], wc_r[_F:_F + _FH],
                   preferred_element_type=jnp.float32)
    acc += jnp.dot(po_r[0, 1], wc_r[_F + _FH:2 * _F],
                   preferred_element_type=jnp.float32)
    acc += jnp.dot(pi_r[0, 0], wc_r[2 * _F:2 * _F + _FH],
                   preferred_element_type=jnp.float32)
    acc += jnp.dot(pi_r[0, 1], wc_r[2 * _F + _FH:3 * _F],
                   preferred_element_type=jnp.float32)
    acc += bc_r[...]
    z = jax.nn.sigmoid(acc[:, :_F])
    h = jnp.tanh(acc[:, _F:])
    y_r[0] = (1.0 - z) * h


def _gate(x, po, pi, wc, bc):
    grid = (4, _N // _BLK)
    return pl.pallas_call(
        _gate_body,
        grid=grid,
        in_specs=[
            pl.BlockSpec((1, _BLK, _F), lambda b, n: (b, n, 0)),
            pl.BlockSpec((1, _NC, _BLK, _FH), lambda b, n: (b, 0, n, 0)),
            pl.BlockSpec((1, _NC, _BLK, _FH), lambda b, n: (b, 0, n, 0)),
            pl.BlockSpec((3 * _F, 2 * _F), lambda b, n: (0, 0)),
            pl.BlockSpec((1, 2 * _F), lambda b, n: (0, 0)),
        ],
        out_specs=pl.BlockSpec((1, _BLK, _F), lambda b, n: (b, n, 0)),
        out_shape=jax.ShapeDtypeStruct((4, _N, _F), jnp.float32),
    )(x, po, pi, wc, bc)


def kernel(x, edge_index, edge_weight, W_z, b_z, W_r, b_r, W_h, b_h):
    row = edge_index[0]
    col = edge_index[1]

    scales = _degscale_kernel(row, col, edge_weight)

    xc = jnp.concatenate(
        [jnp.concatenate([x[b, :, :_FH], x[b, :, _FH:]], axis=0)
         for b in range(4)], axis=0)
    col2 = col.reshape(_E // _CH, _CH)
    row2 = row.reshape(_E // _CH, _CH)

    po = _prop_kernel(xc, row, col2, scales[0])
    pi = _prop_kernel(xc, col, row2, scales[1])

    def _fold(W):
        return jnp.concatenate(
            [W[0, 0, :_F] + W[1, 0, :_F], W[0, 1, :_F], W[1, 1, :_F]], axis=0)

    wc = jnp.concatenate([_fold(W_z), _fold(W_h)], axis=1)
    bc = jnp.concatenate([b_z, b_h]).reshape(1, 2 * _F)

    return _gate(x, po, pi, wc, bc)


# speedup vs baseline: 1.6212x; 1.5202x over previous
import functools

import jax
import jax.numpy as jnp
from jax import lax
from jax.experimental import pallas as pl
from jax.experimental.pallas import tpu as pltpu
from jax.experimental.pallas import tpu_sc as plsc

_N = 10000
_E = 320000
_F = 128
_FH = _F // 2
_NC = 2
_NS = 16
_EPT = _E // _NS
_CH = 80
_NCHUNK = _EPT // _CH
_NTRI = (_NCHUNK - 4) // 3
assert 3 * _NTRI + 4 == _NCHUNK
_RPT = _N // _NS
_NR = 640

_mesh = plsc.VectorSubcoreMesh(core_axis_name="c", subcore_axis_name="s")
_sc_params = pltpu.CompilerParams(needs_layout_passes=False,
                                  use_tc_tiling_on_sc=False)


@functools.partial(
    pl.kernel,
    out_type=jax.ShapeDtypeStruct((2, _E), jnp.float32),
    mesh=_mesh,
    compiler_params=_sc_params,
    scratch_types=[
        pltpu.VMEM((_EPT,), jnp.int32),
        pltpu.VMEM((_EPT,), jnp.float32),
        pltpu.VMEM((_NR, 16), jnp.float32),
        pltpu.VMEM((_EPT,), jnp.float32),
        pltpu.VMEM((5, 128), jnp.int32),
        pltpu.VMEM_SHARED((_NR, 16), jnp.float32),
    ],
)
def _degscale_kernel(row_hbm, col_hbm, wgt_hbm, out_hbm,
                     nod_v, wgt_v, deg_v, scl_v, idx_v, sdeg):
    cid = lax.axis_index("c")
    sid = lax.axis_index("s")

    @pl.when(cid == 0)
    def _load_row():
        pltpu.sync_copy(row_hbm.at[pl.ds(sid * _EPT, _EPT)], nod_v)

    @pl.when(cid == 1)
    def _load_col():
        pltpu.sync_copy(col_hbm.at[pl.ds(sid * _EPT, _EPT)], nod_v)

    pltpu.sync_copy(wgt_hbm.at[pl.ds(sid * _EPT, _EPT)], wgt_v)

    zeros16 = jnp.zeros((16,), jnp.float32)
    one16 = jnp.full((16,), 1.0, jnp.float32)
    iota16 = lax.iota(jnp.int32, 16)

    def _zero(i, carry):
        deg_v[i, pl.ds(0, 16)] = zeros16
        return carry

    lax.fori_loop(0, _NR, _zero, 0)

    @pl.when(sid == 0)
    def _zero_shared():
        pltpu.sync_copy(deg_v, sdeg)

    for k in range(5):
        for u in range(8):
            idx_v[k, pl.ds(16 * u, 16)] = iota16 + (128 * k + 16 * u)

    def _acc(i, carry):
        n16 = nod_v[pl.ds(16 * i, 16)]
        w16 = wgt_v[pl.ds(16 * i, 16)]
        plsc.addupdate_scatter(
            deg_v,
            [lax.shift_right_logical(n16, 4), lax.bitwise_and(n16, 15)], w16)
        return carry

    lax.fori_loop(0, _EPT // 16, _acc, 0)

    plsc.subcore_barrier()

    for k in range(5):
        pltpu.sync_copy(deg_v.at[pl.ds(128 * k, 128)],
                        sdeg.at[idx_v.at[k]], add=True)

    plsc.subcore_barrier()

    pltpu.sync_copy(sdeg, deg_v)

    def _recip(i, carry):
        d = deg_v[i, pl.ds(0, 16)]
        deg_v[i, pl.ds(0, 16)] = one16 / jnp.where(d == 0.0, one16, d)
        return carry

    lax.fori_loop(0, _NR, _recip, 0)

    def _scale(i, carry):
        n16 = nod_v[pl.ds(16 * i, 16)]
        rd = plsc.load_gather(
            deg_v,
            [lax.shift_right_logical(n16, 4), lax.bitwise_and(n16, 15)])
        scl_v[pl.ds(16 * i, 16)] = wgt_v[pl.ds(16 * i, 16)] * rd
        return carry

    lax.fori_loop(0, _EPT // 16, _scale, 0)

    pltpu.sync_copy(scl_v, out_hbm.at[cid, pl.ds(sid * _EPT, _EPT)])


@functools.partial(
    pl.kernel,
    out_type=jax.ShapeDtypeStruct((4, _NC, _N, _FH), jnp.float32),
    mesh=_mesh,
    compiler_params=_sc_params,
    scratch_types=[
        pltpu.VMEM((_EPT,), jnp.int32),
        pltpu.VMEM((_NCHUNK, _CH), jnp.int32),
        pltpu.VMEM((_EPT,), jnp.float32),
        pltpu.VMEM((_CH, _FH), jnp.float32),
        pltpu.VMEM((_CH, _FH), jnp.float32),
        pltpu.VMEM((_CH, _FH), jnp.float32),
        pltpu.VMEM_SHARED((_N, _FH), jnp.float32),
        pltpu.SemaphoreType.DMA,
        pltpu.SemaphoreType.DMA,
        pltpu.SemaphoreType.DMA,
    ],
)
def _prop_kernel(xc, gsrc_hbm, sdst_hbm, scale_hbm,
                 out_hbm,
                 idx_v, dst_v, scale_v, buf0, buf1, buf2,
                 accum, sem0, sem1, sem2):
    cid = lax.axis_index("c")
    sid = lax.axis_index("s")

    pltpu.sync_copy(gsrc_hbm.at[pl.ds(sid * _EPT, _EPT)], idx_v)
    pltpu.sync_copy(sdst_hbm.at[pl.ds(sid * _NCHUNK, _NCHUNK)], dst_v)
    pltpu.sync_copy(scale_hbm.at[pl.ds(sid * _EPT, _EPT)], scale_v)

    zeros16 = jnp.zeros((16,), jnp.float32)

    coff = lax.broadcast(cid * _N, (16,))

    def _coff(i, carry):
        idx_v[pl.ds(16 * i, 16)] = idx_v[pl.ds(16 * i, 16)] + coff
        return carry

    lax.fori_loop(0, _EPT // 16, _coff, 0)

    def _zb(r, carry):
        for f in range(_FH // 16):
            buf0[r, pl.ds(16 * f, 16)] = zeros16
        return carry

    lax.fori_loop(0, _CH, _zb, 0)

    row0 = sid * _RPT
    sizes = []
    off = 0
    while off < _RPT:
        sizes.append(min(_CH, _RPT - off))
        off += _CH

    def _start(j, buf, sem):
        pltpu.async_copy(
            xc.at[idx_v.at[pl.ds(j * _CH, _CH)]], buf, sem)

    def _wait(j, buf, sem):
        pltpu.make_async_copy(
            xc.at[idx_v.at[pl.ds(j * _CH, _CH)]], buf, sem).wait()

    def _proc(j, buf):
        for g in range(_CH // 16):
            sc16 = scale_v[pl.ds(j * _CH + 16 * g, 16)]
            for rr in range(16):
                r = 16 * g + rr
                sv = lax.broadcast(sc16[rr], (16,))
                for f in range(_FH // 16):
                    buf[r, pl.ds(16 * f, 16)] = \
                        buf[r, pl.ds(16 * f, 16)] * sv
        pltpu.sync_copy(buf, accum.at[dst_v.at[j]], add=True)

    def _batch(b, carry):
        off = 0
        for sz in sizes:
            pltpu.sync_copy(buf0.at[pl.ds(0, sz)],
                            accum.at[pl.ds(row0 + off, sz)])
            off += sz
        plsc.subcore_barrier()

        _start(0, buf0, sem0)
        _start(1, buf1, sem1)

        def _tri(g, carry2):
            j = 3 * g
            _wait(j, buf0, sem0)
            _start(j + 2, buf2, sem2)
            _proc(j, buf0)
            _wait(j + 1, buf1, sem1)
            _start(j + 3, buf0, sem0)
            _proc(j + 1, buf1)
            _wait(j + 2, buf2, sem2)
            _start(j + 4, buf1, sem1)
            _proc(j + 2, buf2)
            return carry2

        lax.fori_loop(0, _NTRI, _tri, 0)

        j = 3 * _NTRI
        _wait(j, buf0, sem0)
        _start(j + 2, buf2, sem2)
        _proc(j, buf0)
        _wait(j + 1, buf1, sem1)
        _start(j + 3, buf0, sem0)
        _proc(j + 1, buf1)
        _wait(j + 2, buf2, sem2)
        _proc(j + 2, buf2)
        _wait(j + 3, buf0, sem0)
        _proc(j + 3, buf0)

        boff = lax.broadcast(2 * _N, (16,))

        def _bump(i, c2):
            idx_v[pl.ds(16 * i, 16)] = idx_v[pl.ds(16 * i, 16)] + boff
            return c2

        lax.fori_loop(0, _EPT // 16, _bump, 0)

        plsc.subcore_barrier()

        off = 0
        for sz in sizes:
            pltpu.sync_copy(accum.at[pl.ds(row0 + off, sz)],
                            out_hbm.at[b, cid, pl.ds(row0 + off, sz)])
            off += sz
        lax.fori_loop(0, _CH, _zb, 0)
        return carry

    lax.fori_loop(0, 4, _batch, 0)


_BLK = 400


def _gate_body(x_r, po_r, pi_r, wc_r, bc_r, y_r):
    X = x_r[0]
    acc = jnp.dot(X, wc_r[0:_F], preferred_element_type=jnp.float32)
    acc += jnp.dot(po_r[0, 0], wc_r[_F:_F + _FH],
                   preferred_element_type=jnp.float32)
    acc += jnp.dot(po_r[0, 1], wc_r[_F + _FH:2 * _F],
                   preferred_element_type=jnp.float32)
    acc += jnp.dot(pi_r[0, 0], wc_r[2 * _F:2 * _F + _FH],
                   preferred_element_type=jnp.float32)
    acc += jnp.dot(pi_r[0, 1], wc_r[2 * _F + _FH:3 * _F],
                   preferred_element_type=jnp.float32)
    acc += bc_r[...]
    z = jax.nn.sigmoid(acc[:, :_F])
    h = jnp.tanh(acc[:, _F:])
    y_r[0] = (1.0 - z) * h


def _gate(x, po, pi, wc, bc):
    grid = (4, _N // _BLK)
    return pl.pallas_call(
        _gate_body,
        grid=grid,
        in_specs=[
            pl.BlockSpec((1, _BLK, _F), lambda b, n: (b, n, 0)),
            pl.BlockSpec((1, _NC, _BLK, _FH), lambda b, n: (b, 0, n, 0)),
            pl.BlockSpec((1, _NC, _BLK, _FH), lambda b, n: (b, 0, n, 0)),
            pl.BlockSpec((3 * _F, 2 * _F), lambda b, n: (0, 0)),
            pl.BlockSpec((1, 2 * _F), lambda b, n: (0, 0)),
        ],
        out_specs=pl.BlockSpec((1, _BLK, _F), lambda b, n: (b, n, 0)),
        out_shape=jax.ShapeDtypeStruct((4, _N, _F), jnp.float32),
    )(x, po, pi, wc, bc)


def kernel(x, edge_index, edge_weight, W_z, b_z, W_r, b_r, W_h, b_h):
    row = edge_index[0]
    col = edge_index[1]

    scales = _degscale_kernel(row, col, edge_weight)

    xc = jnp.concatenate(
        [jnp.concatenate([x[b, :, :_FH], x[b, :, _FH:]], axis=0)
         for b in range(4)], axis=0)
    col2 = col.reshape(_E // _CH, _CH)
    row2 = row.reshape(_E // _CH, _CH)

    po = _prop_kernel(xc, row, col2, scales[0])
    pi = _prop_kernel(xc, col, row2, scales[1])

    def _fold(W):
        return jnp.concatenate(
            [W[0, 0, :_F] + W[1, 0, :_F], W[0, 1, :_F], W[1, 1, :_F]], axis=0)

    wc = jnp.concatenate([_fold(W_z), _fold(W_h)], axis=1)
    bc = jnp.concatenate([b_z, b_h]).reshape(1, 2 * _F)

    return _gate(x, po, pi, wc, bc)
